# Initial kernel scaffold; baseline (speedup 1.0000x reference)
import jax
import jax.numpy as jnp
from jax.experimental import pallas as pl


def kernel(points, normals, view_dirs, feature_vectors, phys_points, ray_dirs, cam_loc, W0, b0, W1, b1, W2, b2, W3, b3, W4, b4):
    raise NotImplementedError("write your pallas kernel here")


# speedup vs baseline: 3.9059x; 3.9059x over previous
import functools

import jax
import jax.numpy as jnp
from jax import lax
from jax.experimental import pallas as pl
from jax.experimental.pallas import tpu as pltpu

_N_SAMP = 32
_K = 20
_RADIUS = 9.0
_BIG = 3.0e38


def _embed(x, n_freqs):
    out = [x]
    for i in range(n_freqs):
        f = 2.0 ** i
        out.append(jnp.sin(f * x))
        out.append(jnp.cos(f * x))
    return jnp.concatenate(out, axis=-1)


def _block_kernel(pts_ref, nrm_ref, vdir_ref, fv_ref, gd_ref, sqp_ref, g_ref,
                  rd_ref, cam_ref, w0_ref, b0_ref, w1_ref, b1_ref, w2_ref, b2_ref,
                  w3_ref, b3_ref, w4_ref, b4_ref, out_ref):
    flat = pts_ref[...]
    B = flat.shape[0]

    sq_flat = jnp.sum(flat ** 2, axis=1, keepdims=True)
    qp = jnp.dot(flat, gd_ref[...],
                 preferred_element_type=jnp.float32)
    d2 = sq_flat + sqp_ref[...] - 2.0 * qp
    d2 = jnp.maximum(d2, 0.0)

    def body(_, carry):
        work, c, vstar = carry
        m = jnp.min(work, axis=1, keepdims=True)
        tied = work == m
        cnt = jnp.sum(jnp.where(tied, 1.0, 0.0), axis=1, keepdims=True)
        vstar = jnp.where(c < float(_K), m, vstar)
        work = jnp.where(tied, _BIG, work)
        return work, c + cnt, vstar

    zero = jnp.zeros((B, 1), jnp.float32)
    _, _, vstar = lax.fori_loop(0, _K, body, (d2, zero, zero))

    lt = d2 < vstar
    tie = d2 == vstar
    c_lt = jnp.sum(jnp.where(lt, 1.0, 0.0), axis=1, keepdims=True)
    rank = jnp.where(tie, 1.0, 0.0)
    sh = 1
    while sh < rank.shape[1]:
        shifted = jnp.concatenate(
            [jnp.zeros((B, sh), jnp.float32), rank[:, :-sh]], axis=1)
        rank = rank + shifted
        sh *= 2
    in_topk = lt | (tie & (rank <= (float(_K) - c_lt)))
    validm = in_topk & (d2 < _RADIUS * _RADIUS)
    nnm = validm & (d2 != 0.0)
    mf_top = jnp.where(in_topk, 1.0, 0.0)
    mf_val = jnp.where(validm, 1.0, 0.0)
    mf_nn = jnp.where(nnm, 1.0, 0.0)

    cnt_top = jnp.sum(mf_top, axis=1, keepdims=True)
    cnt_val = jnp.sum(mf_val, axis=1, keepdims=True)

    d = jnp.sqrt(d2)
    w = jnp.maximum(1.0 - d * d2 * (1.0 / _RADIUS ** 3), 0.0)
    wv = mf_val * w

    g = g_ref[...]
    a1 = jnp.dot(wv, g, precision=lax.Precision.HIGHEST,
                 preferred_element_type=jnp.float32)
    a2 = jnp.dot(mf_nn, g, precision=lax.Precision.HIGHEST,
                 preferred_element_type=jnp.float32)

    d0 = jnp.sqrt(sq_flat)
    w0 = jnp.maximum(1.0 - d0 * sq_flat * (1.0 / _RADIUS ** 3), 0.0)
    n_inv = cnt_top - cnt_val
    density = a1[:, 6:7] + n_inv * w0
    smoothed = a1[:, 0:3] / (density + 1e-12)

    num_nn = a2[:, 6:7]
    s1 = a2[:, 0:3] - num_nn * flat
    inv_nn = 1.0 / (num_nn + 1e-12)
    mean = s1 * inv_nn
    s2 = a2[:, 3:6] - 2.0 * flat * a2[:, 0:3] + num_nn * flat * flat
    var = (s2 - 2.0 * mean * s1 + num_nn * mean * mean) * inv_nn

    dirs = smoothed - cam_ref[...]
    sdir = dirs / jnp.sqrt(jnp.sum(dirs * dirs, axis=1, keepdims=True))

    x = jnp.concatenate([
        flat, vdir_ref[...], nrm_ref[...], fv_ref[...],
        _embed(flat, 10), _embed(density, 4), _embed(smoothed, 10),
        _embed(var, 10), _embed(rd_ref[...], 4), _embed(sdir, 4),
    ], axis=1)

    h = jnp.dot(x, w0_ref[...], preferred_element_type=jnp.float32) + b0_ref[...]
    h = jnp.maximum(h, 0.0)
    h = jnp.dot(h, w1_ref[...], preferred_element_type=jnp.float32) + b1_ref[...]
    h = jnp.maximum(h, 0.0)
    h = jnp.dot(h, w2_ref[...], preferred_element_type=jnp.float32) + b2_ref[...]
    h = jnp.maximum(h, 0.0)
    h = jnp.dot(h, w3_ref[...], preferred_element_type=jnp.float32) + b3_ref[...]
    h = jnp.maximum(h, 0.0)
    h = jnp.dot(h, w4_ref[...], preferred_element_type=jnp.float32) + b4_ref[...]
    out_ref[...] = 1.0 / (1.0 + jnp.exp(-h))


@functools.partial(jax.jit, static_argnames=("block_b",))
def _run(points, normals, view_dirs, feature_vectors, phys_points, ray_dirs,
         cam_loc, W0, b0, W1, b1, W2, b2, W3, b3, W4, b4, block_b=128):
    n = points.shape[0]
    p = phys_points.shape[0]
    feat = feature_vectors.shape[1]

    gd = phys_points.T
    sq_p = jnp.sum(phys_points ** 2, -1)[None, :]
    g = jnp.concatenate(
        [phys_points, phys_points * phys_points,
         jnp.ones((p, 1), jnp.float32), jnp.zeros((p, 1), jnp.float32)],
        axis=1)
    rd_rep = jnp.repeat(ray_dirs, _N_SAMP, axis=0)

    wts = [W0.T, W1.T, W2.T, W3.T, W4.T]
    bss = [b.reshape(1, -1) for b in (b0, b1, b2, b3, b4)]

    grid = n // block_b
    row_spec = lambda c: pl.BlockSpec((block_b, c), lambda i: (i, 0))
    full_spec = lambda s: pl.BlockSpec(s, lambda i: (0, 0))

    in_specs = [
        row_spec(3), row_spec(3), row_spec(3), row_spec(feat),
        full_spec((3, p)), full_spec((1, p)), full_spec((p, 8)), row_spec(3),
        full_spec((1, 3)),
    ]
    for wt, bs in zip(wts, bss):
        in_specs.append(full_spec(wt.shape))
        in_specs.append(full_spec(bs.shape))

    out = pl.pallas_call(
        _block_kernel,
        grid=(grid,),
        in_specs=in_specs,
        out_specs=pl.BlockSpec((block_b, 3), lambda i: (i, 0)),
        out_shape=jax.ShapeDtypeStruct((n, 3), jnp.float32),
        compiler_params=pltpu.CompilerParams(
            dimension_semantics=("arbitrary",),
        ),
    )(points, normals, view_dirs, feature_vectors, gd, sq_p, g, rd_rep, cam_loc,
      wts[0], bss[0], wts[1], bss[1], wts[2], bss[2], wts[3], bss[3],
      wts[4], bss[4])
    return out


def kernel(points, normals, view_dirs, feature_vectors, phys_points, ray_dirs,
           cam_loc, W0, b0, W1, b1, W2, b2, W3, b3, W4, b4):
    return _run(points, normals, view_dirs, feature_vectors, phys_points,
                ray_dirs, cam_loc, W0, b0, W1, b1, W2, b2, W3, b3, W4, b4)


# speedup vs baseline: 5.2852x; 1.3531x over previous
import functools

import jax
import jax.numpy as jnp
from jax import lax
from jax.experimental import pallas as pl
from jax.experimental.pallas import tpu as pltpu

_N_SAMP = 32
_K = 20
_RADIUS = 9.0
_BIG = 3.0e38


def _w0_perm():
    def eblk(base, width, nf):
        ident = list(range(base, base + width))
        sin_c, cos_c = [], []
        for i in range(nf):
            off = base + width * (1 + 2 * i)
            sin_c += list(range(off, off + width))
            cos_c += list(range(off + width, off + 2 * width))
        return ident, sin_c, cos_c
    blocks = [eblk(265, 3, 10), eblk(328, 1, 4), eblk(337, 3, 10),
              eblk(400, 3, 10), eblk(463, 3, 4), eblk(490, 3, 4)]
    perm = list(range(0, 265))
    for ident, _, _ in blocks:
        perm += ident
    for _, sin_c, _ in blocks:
        perm += sin_c
    for _, _, cos_c in blocks:
        perm += cos_c
    return perm


def _embed(x, n_freqs):
    out = [x]
    for i in range(n_freqs):
        f = 2.0 ** i
        out.append(jnp.sin(f * x))
        out.append(jnp.cos(f * x))
    return jnp.concatenate(out, axis=-1)


def _block_kernel(pts_ref, nrm_ref, vdir_ref, fv_ref, gd_ref, sqp_ref, g_ref,
                  rd_ref, cam_ref, w0_ref, b0_ref, w1_ref, b1_ref, w2_ref, b2_ref,
                  w3_ref, b3_ref, w4_ref, b4_ref, out_ref):
    flat = pts_ref[...]
    B = flat.shape[0]

    sq_flat = jnp.sum(flat ** 2, axis=1, keepdims=True)
    qp = jnp.dot(flat, gd_ref[...],
                 preferred_element_type=jnp.float32)
    d2 = sq_flat + sqp_ref[...] - 2.0 * qp
    d2 = jnp.maximum(d2, 0.0)

    def body(_, carry):
        work, c, vstar = carry
        m = jnp.min(work, axis=1, keepdims=True)
        tied = work == m
        cnt = jnp.sum(jnp.where(tied, 1.0, 0.0), axis=1, keepdims=True)
        vstar = jnp.where(c < float(_K), m, vstar)
        work = jnp.where(tied, _BIG, work)
        return work, c + cnt, vstar

    zero = jnp.zeros((B, 1), jnp.float32)
    _, _, vstar = lax.fori_loop(0, _K, body, (d2, zero, zero))

    lt = d2 < vstar
    tie = d2 == vstar
    c_lt = jnp.sum(jnp.where(lt, 1.0, 0.0), axis=1, keepdims=True)
    rank = jnp.where(tie, 1.0, 0.0)
    sh = 1
    while sh < rank.shape[1]:
        shifted = jnp.concatenate(
            [jnp.zeros((B, sh), jnp.float32), rank[:, :-sh]], axis=1)
        rank = rank + shifted
        sh *= 2
    in_topk = lt | (tie & (rank <= (float(_K) - c_lt)))
    validm = in_topk & (d2 < _RADIUS * _RADIUS)
    nnm = validm & (d2 != 0.0)
    mf_top = jnp.where(in_topk, 1.0, 0.0)
    mf_val = jnp.where(validm, 1.0, 0.0)
    mf_nn = jnp.where(nnm, 1.0, 0.0)

    cnt_top = jnp.sum(mf_top, axis=1, keepdims=True)
    cnt_val = jnp.sum(mf_val, axis=1, keepdims=True)

    d = jnp.sqrt(d2)
    w = jnp.maximum(1.0 - d * d2 * (1.0 / _RADIUS ** 3), 0.0)
    wv = mf_val * w

    g = g_ref[...]
    a1 = jnp.dot(wv, g, precision=lax.Precision.HIGHEST,
                 preferred_element_type=jnp.float32)
    a2 = jnp.dot(mf_nn, g, precision=lax.Precision.HIGHEST,
                 preferred_element_type=jnp.float32)

    d0 = jnp.sqrt(sq_flat)
    w0 = jnp.maximum(1.0 - d0 * sq_flat * (1.0 / _RADIUS ** 3), 0.0)
    n_inv = cnt_top - cnt_val
    density = a1[:, 6:7] + n_inv * w0
    smoothed = a1[:, 0:3] / (density + 1e-12)

    num_nn = a2[:, 6:7]
    s1 = a2[:, 0:3] - num_nn * flat
    inv_nn = 1.0 / (num_nn + 1e-12)
    mean = s1 * inv_nn
    s2 = a2[:, 3:6] - 2.0 * flat * a2[:, 0:3] + num_nn * flat * flat
    var = (s2 - 2.0 * mean * s1 + num_nn * mean * mean) * inv_nn

    dirs = smoothed - cam_ref[...]
    sdir = dirs / jnp.sqrt(jnp.sum(dirs * dirs, axis=1, keepdims=True))

    rd = rd_ref[...]
    args = jnp.concatenate(
        [flat * (2.0 ** i) for i in range(10)]
        + [density * (2.0 ** i) for i in range(4)]
        + [smoothed * (2.0 ** i) for i in range(10)]
        + [var * (2.0 ** i) for i in range(10)]
        + [rd * (2.0 ** i) for i in range(4)]
        + [sdir * (2.0 ** i) for i in range(4)], axis=1)
    x = jnp.concatenate([
        flat, vdir_ref[...], nrm_ref[...], fv_ref[...],
        flat, density, smoothed, var, rd, sdir,
        jnp.sin(args), jnp.cos(args),
    ], axis=1)

    h = jnp.dot(x, w0_ref[...], preferred_element_type=jnp.float32) + b0_ref[...]
    h = jnp.maximum(h, 0.0)
    h = jnp.dot(h, w1_ref[...], preferred_element_type=jnp.float32) + b1_ref[...]
    h = jnp.maximum(h, 0.0)
    h = jnp.dot(h, w2_ref[...], preferred_element_type=jnp.float32) + b2_ref[...]
    h = jnp.maximum(h, 0.0)
    h = jnp.dot(h, w3_ref[...], preferred_element_type=jnp.float32) + b3_ref[...]
    h = jnp.maximum(h, 0.0)
    h = jnp.dot(h, w4_ref[...], preferred_element_type=jnp.float32) + b4_ref[...]
    out_ref[...] = 1.0 / (1.0 + jnp.exp(-h))


@functools.partial(jax.jit, static_argnames=("block_b",))
def _run(points, normals, view_dirs, feature_vectors, phys_points, ray_dirs,
         cam_loc, W0, b0, W1, b1, W2, b2, W3, b3, W4, b4, block_b=128):
    n = points.shape[0]
    p = phys_points.shape[0]
    feat = feature_vectors.shape[1]

    gd = phys_points.T
    sq_p = jnp.sum(phys_points ** 2, -1)[None, :]
    g = jnp.concatenate(
        [phys_points, phys_points * phys_points,
         jnp.ones((p, 1), jnp.float32), jnp.zeros((p, 1), jnp.float32)],
        axis=1)
    rd_rep = jnp.repeat(ray_dirs, _N_SAMP, axis=0)

    wts = [W0.T[_w0_perm(), :], W1.T, W2.T, W3.T, W4.T]
    bss = [b.reshape(1, -1) for b in (b0, b1, b2, b3, b4)]

    grid = n // block_b
    row_spec = lambda c: pl.BlockSpec((block_b, c), lambda i: (i, 0))
    full_spec = lambda s: pl.BlockSpec(s, lambda i: (0, 0))

    in_specs = [
        row_spec(3), row_spec(3), row_spec(3), row_spec(feat),
        full_spec((3, p)), full_spec((1, p)), full_spec((p, 8)), row_spec(3),
        full_spec((1, 3)),
    ]
    for wt, bs in zip(wts, bss):
        in_specs.append(full_spec(wt.shape))
        in_specs.append(full_spec(bs.shape))

    out = pl.pallas_call(
        _block_kernel,
        grid=(grid,),
        in_specs=in_specs,
        out_specs=pl.BlockSpec((block_b, 3), lambda i: (i, 0)),
        out_shape=jax.ShapeDtypeStruct((n, 3), jnp.float32),
        compiler_params=pltpu.CompilerParams(
            dimension_semantics=("arbitrary",),
        ),
    )(points, normals, view_dirs, feature_vectors, gd, sq_p, g, rd_rep, cam_loc,
      wts[0], bss[0], wts[1], bss[1], wts[2], bss[2], wts[3], bss[3],
      wts[4], bss[4])
    return out


def kernel(points, normals, view_dirs, feature_vectors, phys_points, ray_dirs,
           cam_loc, W0, b0, W1, b1, W2, b2, W3, b3, W4, b4):
    return _run(points, normals, view_dirs, feature_vectors, phys_points,
                ray_dirs, cam_loc, W0, b0, W1, b1, W2, b2, W3, b3, W4, b4)
